# baseline (speedup 1.0000x reference)
import jax
import jax.numpy as jnp
from jax import lax
from jax.experimental import pallas as pl
from jax.experimental.pallas import tpu as pltpu
from jax.experimental.pallas import tpu_sc as plsc

_BATCH = 16384
_IN = 40
_L = 16
_EMB = 32

_info = plsc.get_sparse_core_info()
_NCORES = 1
_NW = _NCORES * _info.num_subcores
_RPW = _BATCH // _NW
_NGRP = _RPW // _L


def _sc_body(ids_hbm, w0_hbm, w12_hbm, emb_hbm, ids_v, w0_v, w12_v, out_v):
    wid = lax.axis_index("s") * _NCORES + lax.axis_index("c")
    base = wid * _RPW

    pltpu.sync_copy(ids_hbm.at[pl.ds(base, _RPW)], ids_v)
    pltpu.sync_copy(w0_hbm, w0_v)
    pltpu.sync_copy(w12_hbm, w12_v)

    col1 = jnp.full((_L,), 1, jnp.int32)
    col2 = jnp.full((_L,), 2, jnp.int32)
    jcol = lax.iota(jnp.int32, _L)

    @plsc.parallel_loop(0, _NGRP, unroll=1)
    def _grp(g):
        rows = lax.iota(jnp.int32, _L) + g * _L
        i0 = plsc.load_gather(ids_v, [rows, jnp.zeros((_L,), jnp.int32)]).astype(jnp.int32)
        i1 = plsc.load_gather(ids_v, [rows, col1]).astype(jnp.int32)
        i2 = plsc.load_gather(ids_v, [rows, col2]).astype(jnp.int32)
        b0 = jnp.where((i0 >= 0) & (i0 < 64), i0, 64)
        b1 = jnp.where((i1 >= 0) & (i1 < 32), i1, 32)
        b2 = jnp.where((i2 >= 0) & (i2 < 16), i2, 16)
        b12 = b1 * 17 + b2
        obase = rows * _EMB
        for j in range(_L):
            jv = jnp.full((_L,), j, jnp.int32)
            v0 = plsc.load_gather(w0_v, [b0, jv])
            plsc.store_scatter(out_v, [obase + j], v0)
            v12 = plsc.load_gather(w12_v, [b12, jv])
            plsc.store_scatter(out_v, [obase + (16 + j)], v12)

    pltpu.sync_copy(out_v, emb_hbm.at[pl.ds(base * _EMB, _RPW * _EMB)])


def _tc_pass_body(in_ref, out_ref):
    out_ref[...] = in_ref[:, 3:_IN]


def kernel(inputs, W0, W1, W2):
    W12 = jnp.concatenate(
        [jnp.repeat(W1, 17, axis=0), jnp.tile(W2, (33, 1))], axis=1)
    ids8 = inputs[:, :8]

    mesh = plsc.VectorSubcoreMesh(core_axis_name="c", subcore_axis_name="s",
                                  num_cores=_NCORES)
    sc_run = pl.kernel(
        _sc_body,
        out_type=jax.ShapeDtypeStruct((_BATCH * _EMB,), jnp.float32),
        mesh=mesh,
        compiler_params=pltpu.CompilerParams(use_tc_tiling_on_sc=False,
                                             needs_layout_passes=False),
        scratch_types=[
            pltpu.VMEM((_RPW, 8), jnp.float32),
            pltpu.VMEM((65, 16), jnp.float32),
            pltpu.VMEM((561, 16), jnp.float32),
            pltpu.VMEM((_RPW * _EMB,), jnp.float32),
        ],
    )
    emb = sc_run(ids8, W0, W12).reshape(_BATCH, _EMB)

    blk = 2048
    passed = pl.pallas_call(
        _tc_pass_body,
        out_shape=jax.ShapeDtypeStruct((_BATCH, _IN - 3), jnp.float32),
        grid=(_BATCH // blk,),
        in_specs=[pl.BlockSpec((blk, _IN), lambda i: (i, 0))],
        out_specs=pl.BlockSpec((blk, _IN - 3), lambda i: (i, 0)),
    )(inputs)

    return jnp.concatenate([passed, emb], axis=1)
